# baseline (speedup 1.0000x reference)
import jax
import jax.numpy as jnp
from jax.experimental import pallas as pl
from jax.experimental.pallas import tpu as pltpu

B, T, F, D = 1024, 50, 26, 32
TF = T * F
TFBLK = 20
NBLK = TFBLK * D
BBLK = 256


def _bias_body(t_ref, f_ref, bv_ref, out_ref):
    t = t_ref[...]
    f = f_ref[...]
    bv = bv_ref[...]
    out_ref[...] = t[:, None, :] + f[None, :, :] + bv[0][None, None, :]


def _main_body(x_ref, wt_ref, bias_ref, out_ref, m_ref):
    @pl.when(pl.program_id(0) == 0)
    def _():
        row = jax.lax.broadcasted_iota(jnp.int32, (TFBLK, NBLK), 0)
        col = jax.lax.broadcasted_iota(jnp.int32, (TFBLK, NBLK), 1)
        wt = wt_ref[...]
        m_ref[...] = jnp.where(col // D == row, wt, 0.0)

    y = jax.lax.dot_general(
        x_ref[0], m_ref[...],
        dimension_numbers=(((0,), (0,)), ((), ())),
        preferred_element_type=jnp.float32,
    )
    out_ref[...] = y + bias_ref[...]


def kernel(x, W_val, b_val, time_embed, feat_embed):
    bias3 = pl.pallas_call(
        _bias_body,
        out_shape=jax.ShapeDtypeStruct((T, F, D), jnp.float32),
    )(time_embed, feat_embed, b_val.reshape(1, D))
    bias_flat = bias3.reshape(1, TF * D)

    xt = x.reshape(B, TF).T.reshape(TF // TFBLK, TFBLK, B)
    w_tiled = jnp.tile(W_val, (1, TFBLK))

    out_flat = pl.pallas_call(
        _main_body,
        grid=(TF // TFBLK,),
        in_specs=[
            pl.BlockSpec((1, TFBLK, B), lambda j: (j, 0, 0)),
            pl.BlockSpec((1, NBLK), lambda j: (0, 0)),
            pl.BlockSpec((1, NBLK), lambda j: (0, j)),
        ],
        out_specs=pl.BlockSpec((B, NBLK), lambda j: (0, j)),
        out_shape=jax.ShapeDtypeStruct((B, TF * D), jnp.float32),
        scratch_shapes=[pltpu.VMEM((TFBLK, NBLK), jnp.float32)],
    )(xt, w_tiled, bias_flat)

    return out_flat.reshape(B, TF, D)


# speedup vs baseline: 1.0999x; 1.0999x over previous
import jax
import jax.numpy as jnp
from jax.experimental import pallas as pl
from jax.experimental.pallas import tpu as pltpu

B, T, F, D = 1024, 50, 26, 32
TF = T * F
TFBLK = 20
NBLK = TFBLK * D
NGRP = TF // TFBLK
BBLK = 64


def _bias_body(t_ref, f_ref, bv_ref, out_ref):
    t = t_ref[...]
    f = f_ref[...]
    bv = bv_ref[...]
    out_ref[...] = t[:, None, :] + f[None, :, :] + bv[0][None, None, :]


def _main_body(x_ref, wt_ref, bias_ref, out_ref, m_ref):
    @pl.when(pl.program_id(0) == 0)
    def _():
        row = jax.lax.broadcasted_iota(jnp.int32, (TFBLK, NBLK), 0)
        col = jax.lax.broadcasted_iota(jnp.int32, (TFBLK, NBLK), 1)
        wt = wt_ref[...]
        m_ref[...] = jnp.where(col // D == row, wt, 0.0)

    bias = bias_ref[...]
    m = m_ref[...]
    for j in range(NGRP):
        y = jax.lax.dot_general(
            x_ref[:, TFBLK * j:TFBLK * (j + 1)], m,
            dimension_numbers=(((1,), (0,)), ((), ())),
            preferred_element_type=jnp.float32,
        )
        out_ref[:, NBLK * j:NBLK * (j + 1)] = y + bias[:, NBLK * j:NBLK * (j + 1)]


def kernel(x, W_val, b_val, time_embed, feat_embed):
    bias3 = pl.pallas_call(
        _bias_body,
        out_shape=jax.ShapeDtypeStruct((T, F, D), jnp.float32),
    )(time_embed, feat_embed, b_val.reshape(1, D))
    bias_flat = bias3.reshape(1, TF * D)

    x2 = x.reshape(B, TF)
    w_tiled = jnp.tile(W_val, (1, TFBLK))

    out_flat = pl.pallas_call(
        _main_body,
        grid=(B // BBLK,),
        in_specs=[
            pl.BlockSpec((BBLK, TF), lambda i: (i, 0)),
            pl.BlockSpec((1, NBLK), lambda i: (0, 0)),
            pl.BlockSpec((1, TF * D), lambda i: (0, 0)),
        ],
        out_specs=pl.BlockSpec((BBLK, TF * D), lambda i: (i, 0)),
        out_shape=jax.ShapeDtypeStruct((B, TF * D), jnp.float32),
        scratch_shapes=[pltpu.VMEM((TFBLK, NBLK), jnp.float32)],
    )(x2, w_tiled, bias_flat)

    return out_flat.reshape(B, TF, D)
